# baseline (speedup 1.0000x reference)
import functools

import jax
import jax.numpy as jnp
from jax import lax
from jax.experimental import pallas as pl
from jax.experimental.pallas import tpu as pltpu
from jax.experimental.pallas import tpu_sc as plsc

NUM_CODES = 8192
DIM = 32
N_TOKENS = 8192
MBLK = 1024
NBLK = 2048
M_CHUNKS = N_TOKENS // MBLK
N_CHUNKS = NUM_CODES // NBLK
COMMIT = 0.25
LOSS_SCALE = (1.0 + COMMIT) / (N_TOKENS * DIM)


def _vq_body(x_ref, e_ref, idx_ref, loss_ref):
    m = pl.program_id(0)
    x = x_ref[...]
    xm = x * (-2.0)
    x2 = jnp.sum(x * x, axis=1, keepdims=True)
    iif = lax.broadcasted_iota(jnp.int32, (1, NBLK), 1).astype(jnp.float32)

    bd = bi = bx = None
    for n in range(N_CHUNKS):
        e = e_ref[:, n * NBLK:(n + 1) * NBLK]
        e2 = jnp.sum(e * e, axis=0, keepdims=True)
        xe2 = jnp.dot(xm, e, preferred_element_type=jnp.float32)
        d = (x2 + xe2) + e2
        lmin = jnp.min(d, axis=1, keepdims=True)
        lminb = lmin.astype(jnp.bfloat16).astype(jnp.float32)
        lidxf = jnp.min(jnp.where(d == lmin, iif, jnp.float32(3e38)),
                        axis=1, keepdims=True)
        lidx = lidxf.astype(jnp.int32) + n * NBLK
        if n == 0:
            bd, bi, bx = lminb, lidx, lmin
        else:
            upd = lmin < bd
            bd = jnp.where(upd, lminb, bd)
            bi = jnp.where(upd, lidx, bi)
            bx = jnp.where(upd, lmin, bx)

    idx_ref[...] = bi

    @pl.when(m == 0)
    def _():
        loss_ref[...] = jnp.zeros_like(loss_ref)

    loss_ref[...] += (jnp.sum(bx) * LOSS_SCALE).reshape(1, 1)


_vq_call = pl.pallas_call(
    _vq_body,
    grid=(M_CHUNKS,),
    in_specs=[
        pl.BlockSpec((MBLK, DIM), lambda m: (m, 0)),
        pl.BlockSpec((DIM, NUM_CODES), lambda m: (0, 0)),
    ],
    out_specs=[
        pl.BlockSpec((MBLK, 1), lambda m: (m, 0)),
        pl.BlockSpec((1, 1), lambda m: (0, 0)),
    ],
    out_shape=[
        jax.ShapeDtypeStruct((N_TOKENS, 1), jnp.int32),
        jax.ShapeDtypeStruct((1, 1), jnp.float32),
    ],
    compiler_params=pltpu.CompilerParams(
        dimension_semantics=("parallel",)),
)


_NC, _NS = 2, 16
_NW = _NC * _NS
_B_PER_W = N_TOKENS // _NW


@functools.cache
def _sc_gather_call():
    @functools.partial(
        pl.kernel,
        mesh=plsc.VectorSubcoreMesh(core_axis_name="c", subcore_axis_name="s"),
        out_type=jax.ShapeDtypeStruct((N_TOKENS, DIM), jnp.float32),
        scratch_types=[
            pltpu.VMEM((_B_PER_W,), jnp.int32),
            pltpu.VMEM((_B_PER_W, DIM), jnp.float32),
            pltpu.SemaphoreType.DMA,
        ],
        compiler_params=pltpu.CompilerParams(use_tc_tiling_on_sc=False),
    )
    def _sc_gather(table_hbm, idx_hbm, out_hbm, idx_v, rows_v, sem):
        wid = lax.axis_index("s") * _NC + lax.axis_index("c")
        base = wid * _B_PER_W
        pltpu.sync_copy(idx_hbm.at[pl.ds(base, _B_PER_W)], idx_v)
        pltpu.async_copy(table_hbm.at[idx_v], rows_v, sem).wait()
        pltpu.sync_copy(rows_v, out_hbm.at[pl.ds(base, _B_PER_W)])

    return _sc_gather


def kernel(inputs, embeddings):
    x = inputs.astype(jnp.float32).reshape(-1, DIM)
    idx2d, loss = _vq_call(x, embeddings)
    idx = idx2d.reshape(N_TOKENS)
    table = embeddings.T
    q = _sc_gather_call()(table, idx)
    quantized = q.reshape(inputs.shape).astype(inputs.dtype)
    return quantized, idx.reshape(inputs.shape[:-1]), loss.reshape(())


# speedup vs baseline: 1.0049x; 1.0049x over previous
import functools

import jax
import jax.numpy as jnp
from jax import lax
from jax.experimental import pallas as pl
from jax.experimental.pallas import tpu as pltpu
from jax.experimental.pallas import tpu_sc as plsc

NUM_CODES = 8192
DIM = 32
N_TOKENS = 8192
MBLK = 1024
NBLK = 2048
M_CHUNKS = N_TOKENS // MBLK
N_CHUNKS = NUM_CODES // NBLK
COMMIT = 0.25
LOSS_SCALE = (1.0 + COMMIT) / (N_TOKENS * DIM)


def _vq_body(x_ref, e_ref, eblk_ref, idx_ref, loss_ref, et_ref):
    m = pl.program_id(0)
    et_ref[...] = jnp.transpose(eblk_ref[...], (1, 0))
    x = x_ref[...]
    xm = x * (-2.0)
    x2 = jnp.sum(x * x, axis=1, keepdims=True)
    iif = lax.broadcasted_iota(jnp.int32, (1, NBLK), 1).astype(jnp.float32)

    bd = bi = bx = None
    for n in range(N_CHUNKS):
        e = e_ref[:, n * NBLK:(n + 1) * NBLK]
        e2 = jnp.sum(e * e, axis=0, keepdims=True)
        xe2 = jnp.dot(xm, e, preferred_element_type=jnp.float32)
        d = (x2 + xe2) + e2
        lmin = jnp.min(d, axis=1, keepdims=True)
        lminb = lmin.astype(jnp.bfloat16).astype(jnp.float32)
        lidxf = jnp.min(jnp.where(d == lmin, iif, jnp.float32(3e38)),
                        axis=1, keepdims=True)
        lidx = lidxf.astype(jnp.int32) + n * NBLK
        if n == 0:
            bd, bi, bx = lminb, lidx, lmin
        else:
            upd = lmin < bd
            bd = jnp.where(upd, lminb, bd)
            bi = jnp.where(upd, lidx, bi)
            bx = jnp.where(upd, lmin, bx)

    idx_ref[...] = bi

    @pl.when(m == 0)
    def _():
        loss_ref[...] = jnp.zeros_like(loss_ref)

    loss_ref[...] += (jnp.sum(bx) * LOSS_SCALE).reshape(1, 1)


_vq_call = pl.pallas_call(
    _vq_body,
    grid=(M_CHUNKS,),
    in_specs=[
        pl.BlockSpec((MBLK, DIM), lambda m: (m, 0)),
        pl.BlockSpec((DIM, NUM_CODES), lambda m: (0, 0)),
        pl.BlockSpec((DIM, MBLK), lambda m: (0, m)),
    ],
    out_specs=[
        pl.BlockSpec((MBLK, 1), lambda m: (m, 0)),
        pl.BlockSpec((1, 1), lambda m: (0, 0)),
        pl.BlockSpec((MBLK, DIM), lambda m: (m, 0)),
    ],
    out_shape=[
        jax.ShapeDtypeStruct((N_TOKENS, 1), jnp.int32),
        jax.ShapeDtypeStruct((1, 1), jnp.float32),
        jax.ShapeDtypeStruct((NUM_CODES, DIM), jnp.float32),
    ],
    compiler_params=pltpu.CompilerParams(
        dimension_semantics=("parallel",)),
)


_NC, _NS = 2, 16
_NW = _NC * _NS
_B_PER_W = N_TOKENS // _NW


@functools.cache
def _sc_gather_call():
    @functools.partial(
        pl.kernel,
        mesh=plsc.VectorSubcoreMesh(core_axis_name="c", subcore_axis_name="s"),
        out_type=jax.ShapeDtypeStruct((N_TOKENS, DIM), jnp.float32),
        scratch_types=[
            pltpu.VMEM((_B_PER_W,), jnp.int32),
            pltpu.VMEM((_B_PER_W, DIM), jnp.float32),
            pltpu.SemaphoreType.DMA,
        ],
        compiler_params=pltpu.CompilerParams(use_tc_tiling_on_sc=False),
    )
    def _sc_gather(table_hbm, idx_hbm, out_hbm, idx_v, rows_v, sem):
        wid = lax.axis_index("s") * _NC + lax.axis_index("c")
        base = wid * _B_PER_W
        pltpu.sync_copy(idx_hbm.at[pl.ds(base, _B_PER_W)], idx_v)
        pltpu.async_copy(table_hbm.at[idx_v], rows_v, sem).wait()
        pltpu.sync_copy(rows_v, out_hbm.at[pl.ds(base, _B_PER_W)])

    return _sc_gather


def kernel(inputs, embeddings):
    x = inputs.astype(jnp.float32).reshape(-1, DIM)
    idx2d, loss, table = _vq_call(x, embeddings, embeddings)
    idx = idx2d.reshape(N_TOKENS)
    q = _sc_gather_call()(table, idx)
    quantized = q.reshape(inputs.shape).astype(inputs.dtype)
    return quantized, idx.reshape(inputs.shape[:-1]), loss.reshape(())
